# baseline (speedup 1.0000x reference)
import functools

import jax
import jax.numpy as jnp
from jax import lax
from jax.experimental import pallas as pl
from jax.experimental.pallas import tpu as pltpu
from jax.experimental.pallas import tpu_sc as plsc

BATCH = 16384
K = 1000
BR = 512
NB = BATCH // BR

L = 16
NC = 2
NS = 16
NW = NC * NS
CHUNK1 = BATCH // NS
CHUNK2 = BATCH // NW
KPAD = 1024
SW = 128
R1 = CHUNK1 // SW


def _sc_body(lbl_hbm, c1_hbm, g_hbm, cr_hbm, d_hbm,
             lbl1_v, ones_v, bins_v, bins_sh,
             lbl2_v, cidx_v, g_v, cr_v, d_v, sem):
    cid = lax.axis_index("c")
    sid = lax.axis_index("s")
    wid = sid * NC + cid

    iota = lax.iota(jnp.int32, L)
    ones16 = jnp.ones((L,), jnp.float32)
    zeros16 = jnp.zeros((L,), jnp.float32)

    base2 = wid * CHUNK2
    pltpu.sync_copy(lbl_hbm.at[pl.ds(base2, CHUNK2)], lbl2_v)

    def _cidx(j, carry):
        l16 = lbl2_v[pl.ds(j * L, L)]
        i16 = base2 + j * L + iota
        cidx_v[pl.ds(j * L, L)] = i16 * K + l16
        return carry
    lax.fori_loop(0, CHUNK2 // L, _cidx, 0)

    gcps = [
        pltpu.async_copy(
            c1_hbm.at[cidx_v.at[pl.ds(t * SW, SW)]],
            g_v.at[pl.ds(t * SW, SW)],
            sem,
        )
        for t in range(CHUNK2 // SW)
    ]

    def _fill(j, carry):
        bins_v[pl.ds(j * L, L)] = zeros16
        return carry
    lax.fori_loop(0, KPAD // L, _fill, 0)

    def _fill1(j, carry):
        ones_v[pl.ds(j * L, L)] = ones16
        return carry
    lax.fori_loop(0, SW // L, _fill1, 0)

    base1 = sid * CHUNK1
    for j in range(R1):
        pltpu.sync_copy(lbl_hbm.at[pl.ds(base1 + j * SW, SW)], lbl1_v.at[j])

    @pl.when(sid == 0)
    def _():
        pltpu.sync_copy(bins_v, bins_sh)

    plsc.subcore_barrier()
    for j in range(R1):
        pltpu.sync_copy(ones_v, bins_sh.at[lbl1_v.at[j]], add=True)
    plsc.subcore_barrier()

    pltpu.sync_copy(bins_sh, bins_v)

    for t in range(CHUNK2 // SW):
        pltpu.sync_copy(
            bins_sh.at[lbl2_v.at[pl.ds(t * SW, SW)]],
            cr_v.at[pl.ds(t * SW, SW)],
        )

    for cp in gcps:
        cp.wait()

    pltpu.sync_copy(g_v, g_hbm.at[pl.ds(base2, CHUNK2)])
    pltpu.sync_copy(cr_v, cr_hbm.at[pl.ds(base2, CHUNK2)])

    @pl.when((cid == 0) & (sid == 0))
    def _():
        def _dd(j, a):
            return a + jnp.where(bins_v[pl.ds(j * L, L)] > 0.0, 1.0, 0.0)
        d_v[...] = lax.fori_loop(0, KPAD // L, _dd, zeros16)
        pltpu.sync_copy(d_v, d_hbm)


_sc_stats = functools.partial(
    pl.kernel,
    mesh=plsc.VectorSubcoreMesh(core_axis_name="c", subcore_axis_name="s"),
    out_type=[
        jax.ShapeDtypeStruct((BATCH,), jnp.float32),
        jax.ShapeDtypeStruct((BATCH,), jnp.float32),
        jax.ShapeDtypeStruct((L,), jnp.float32),
    ],
    scratch_types=[
        pltpu.VMEM((R1, SW), jnp.int32),
        pltpu.VMEM((SW,), jnp.float32),
        pltpu.VMEM((KPAD,), jnp.float32),
        pltpu.VMEM_SHARED((KPAD,), jnp.float32),
        pltpu.VMEM((CHUNK2,), jnp.int32),
        pltpu.VMEM((CHUNK2,), jnp.int32),
        pltpu.VMEM((CHUNK2,), jnp.float32),
        pltpu.VMEM((CHUNK2,), jnp.float32),
        pltpu.VMEM((L,), jnp.float32),
        pltpu.SemaphoreType.DMA,
    ],
)(_sc_body)


def _tc_body(c_ref, g_ref, cr_ref, d_ref, loss_ref, acc_s):
    k = pl.program_id(0)

    @pl.when(k == 0)
    def _():
        acc_s[...] = jnp.zeros_like(acc_s)

    cb = c_ref[...]
    m = jnp.max(cb, axis=1, keepdims=True)
    s = jnp.sum(jnp.exp(cb - m), axis=1, keepdims=True)
    lse = m + jnp.log(s)
    val = (lse - g_ref[...]) / cr_ref[...]
    acc_s[...] += jnp.sum(val, keepdims=True)

    @pl.when(k == NB - 1)
    def _():
        loss_ref[...] = acc_s[...] / jnp.sum(d_ref[...], keepdims=True)


def kernel(c, pseudo_label):
    lbl = pseudo_label.astype(jnp.int32)
    c1 = c.reshape(BATCH * K)
    g, cr, dv = _sc_stats(lbl, c1)

    out = pl.pallas_call(
        _tc_body,
        grid=(NB,),
        in_specs=[
            pl.BlockSpec((BR, K), lambda k: (k, 0)),
            pl.BlockSpec((BR, 1), lambda k: (k, 0)),
            pl.BlockSpec((BR, 1), lambda k: (k, 0)),
            pl.BlockSpec((1, L), lambda k: (0, 0)),
        ],
        out_specs=pl.BlockSpec((1, 1), lambda k: (0, 0)),
        out_shape=jax.ShapeDtypeStruct((1, 1), jnp.float32),
        scratch_shapes=[pltpu.VMEM((1, 1), jnp.float32)],
    )(c, g.reshape(BATCH, 1), cr.reshape(BATCH, 1), dv.reshape(1, L))
    return out[0, 0]


# speedup vs baseline: 1.6219x; 1.6219x over previous
import functools

import jax
import jax.numpy as jnp
from jax import lax
from jax.experimental import pallas as pl
from jax.experimental.pallas import tpu as pltpu
from jax.experimental.pallas import tpu_sc as plsc

BATCH = 16384
K = 1000
BR = 512
NB = BATCH // BR

L = 16
NC = 2
NS = 16
NW = NC * NS
CHUNK1 = BATCH // NS
CHUNK2 = BATCH // NW
KPAD = 1024
SW = 128
R1 = CHUNK1 // SW


def _sc_body(lbl_hbm, cr_hbm, d_hbm,
             lbl1_v, ones_v, bins_v, bins_sh,
             lbl2_v, cr_v, d_v):
    cid = lax.axis_index("c")
    sid = lax.axis_index("s")
    wid = sid * NC + cid

    ones16 = jnp.ones((L,), jnp.float32)
    zeros16 = jnp.zeros((L,), jnp.float32)

    base2 = wid * CHUNK2
    pltpu.sync_copy(lbl_hbm.at[pl.ds(base2, CHUNK2)], lbl2_v)

    def _fill(j, carry):
        bins_v[pl.ds(j * L, L)] = zeros16
        return carry
    lax.fori_loop(0, KPAD // L, _fill, 0)

    def _fill1(j, carry):
        ones_v[pl.ds(j * L, L)] = ones16
        return carry
    lax.fori_loop(0, SW // L, _fill1, 0)

    base1 = sid * CHUNK1
    for j in range(R1):
        pltpu.sync_copy(lbl_hbm.at[pl.ds(base1 + j * SW, SW)], lbl1_v.at[j])

    @pl.when(sid == 0)
    def _():
        pltpu.sync_copy(bins_v, bins_sh)

    plsc.subcore_barrier()
    for j in range(R1):
        pltpu.sync_copy(ones_v, bins_sh.at[lbl1_v.at[j]], add=True)
    plsc.subcore_barrier()

    pltpu.sync_copy(bins_sh, bins_v)

    for t in range(CHUNK2 // SW):
        pltpu.sync_copy(
            bins_sh.at[lbl2_v.at[pl.ds(t * SW, SW)]],
            cr_v.at[pl.ds(t * SW, SW)],
        )
    pltpu.sync_copy(cr_v, cr_hbm.at[pl.ds(base2, CHUNK2)])

    @pl.when((cid == 0) & (sid == 0))
    def _():
        def _dd(j, a):
            return a + jnp.where(bins_v[pl.ds(j * L, L)] > 0.0, 1.0, 0.0)
        d_v[...] = lax.fori_loop(0, KPAD // L, _dd, zeros16)
        pltpu.sync_copy(d_v, d_hbm)


_sc_stats = functools.partial(
    pl.kernel,
    mesh=plsc.VectorSubcoreMesh(core_axis_name="c", subcore_axis_name="s"),
    out_type=[
        jax.ShapeDtypeStruct((BATCH,), jnp.float32),
        jax.ShapeDtypeStruct((L,), jnp.float32),
    ],
    scratch_types=[
        pltpu.VMEM((R1, SW), jnp.int32),
        pltpu.VMEM((SW,), jnp.float32),
        pltpu.VMEM((KPAD,), jnp.float32),
        pltpu.VMEM_SHARED((KPAD,), jnp.float32),
        pltpu.VMEM((CHUNK2,), jnp.int32),
        pltpu.VMEM((CHUNK2,), jnp.float32),
        pltpu.VMEM((L,), jnp.float32),
    ],
)(_sc_body)


def _tc_body(lbl_ref, cr_ref, d_ref, c_ref, loss_ref, acc_s):
    k = pl.program_id(0)

    @pl.when(k == 0)
    def _():
        acc_s[...] = jnp.zeros_like(acc_s)

    cb = c_ref[...]
    m = jnp.max(cb, axis=1, keepdims=True)
    s = jnp.sum(jnp.exp(cb - m), axis=1, keepdims=True)
    lse = m + jnp.log(s)

    onehot = jax.lax.broadcasted_iota(jnp.int32, (BR, K), 1) == lbl_ref[...]
    g = jnp.sum(jnp.where(onehot, cb, 0.0), axis=1, keepdims=True)
    val = (lse - g) / cr_ref[...]
    acc_s[...] += jnp.sum(val, keepdims=True)

    @pl.when(k == NB - 1)
    def _():
        loss_ref[...] = acc_s[...] / jnp.sum(d_ref[...], keepdims=True)


def kernel(c, pseudo_label):
    lbl = pseudo_label.astype(jnp.int32)
    cr, dv = _sc_stats(lbl)

    out = pl.pallas_call(
        _tc_body,
        grid=(NB,),
        in_specs=[
            pl.BlockSpec((BR, 1), lambda k: (k, 0)),
            pl.BlockSpec((BR, 1), lambda k: (k, 0)),
            pl.BlockSpec((1, L), lambda k: (0, 0)),
            pl.BlockSpec((BR, K), lambda k: (k, 0)),
        ],
        out_specs=pl.BlockSpec((1, 1), lambda k: (0, 0)),
        out_shape=jax.ShapeDtypeStruct((1, 1), jnp.float32),
        scratch_shapes=[pltpu.VMEM((1, 1), jnp.float32)],
    )(lbl.reshape(BATCH, 1), cr.reshape(BATCH, 1), dv.reshape(1, L), c)
    return out[0, 0]
